# Initial kernel scaffold; baseline (speedup 1.0000x reference)
import jax
import jax.numpy as jnp
from jax.experimental import pallas as pl


def kernel(im1w_score):
    raise NotImplementedError("write your pallas kernel here")


# speedup vs baseline: 84.6691x; 84.6691x over previous
import numpy as np
import jax
import jax.numpy as jnp
from jax import lax
from jax.experimental import pallas as pl
from jax.experimental.pallas import tpu as pltpu

_K = 512
_BORDER = 8
_R_NMS = 2
_R_G = 3
_GAUSS = np.exp(-2.0 * (np.arange(-_R_G, _R_G + 1) ** 2)).astype(np.float32)
_ONE_BITS = 0x3F800000


def _shift_rows(a, d):
    h, w = a.shape
    if d == 0:
        return a
    if d > 0:
        return jnp.concatenate([jnp.zeros((d, w), a.dtype), a[: h - d]], axis=0)
    return jnp.concatenate([a[-d:], jnp.zeros((-d, w), a.dtype)], axis=0)


def _shift_cols(a, d):
    h, w = a.shape
    if d == 0:
        return a
    if d > 0:
        return jnp.concatenate([jnp.zeros((h, d), a.dtype), a[:, : w - d]], axis=1)
    return jnp.concatenate([a[:, -d:], jnp.zeros((h, -d), a.dtype)], axis=1)


def _body(x_ref, out_ref, tmask_ref, topkv_ref):
    h, w = x_ref.shape[1], x_ref.shape[2]
    x = x_ref[0]

    row = lax.broadcasted_iota(jnp.int32, (h, w), 0)
    col = lax.broadcasted_iota(jnp.int32, (h, w), 1)
    inb = (row >= _BORDER) & (row < h - _BORDER) & \
          (col >= _BORDER) & (col < w - _BORDER)
    x = jnp.where(inb, x, 0.0)
    xt = jnp.where(x < 0.0, 0.0, x)

    m1 = xt
    for d in range(1, _R_NMS + 1):
        m1 = jnp.maximum(m1, jnp.maximum(_shift_cols(xt, d), _shift_cols(xt, -d)))
    mx = m1
    for d in range(1, _R_NMS + 1):
        mx = jnp.maximum(mx, jnp.maximum(_shift_rows(m1, d), _shift_rows(m1, -d)))
    y = jnp.where(xt >= mx, x, 0.0)
    topkv_ref[0] = y

    yi = lax.bitcast_convert_type(y, jnp.int32)

    def _cnt_ge(t):
        return jnp.sum((yi >= t).astype(jnp.int32))

    def _bis(_, lo_hi):
        lo, hi = lo_hi
        mid = (lo + hi) // 2
        take = _cnt_ge(mid) >= _K
        return jnp.where(take, mid, lo), jnp.where(take, hi, mid)

    t_star, _ = lax.fori_loop(0, 30, _bis, (jnp.int32(0), jnp.int32(_ONE_BITS)))

    cnt_gt = jnp.sum((yi > t_star).astype(jnp.int32))
    need = _K - cnt_gt
    ties = yi == t_star
    flat = row * w + col

    def _tie_cnt_le(m):
        return jnp.sum((ties & (flat <= m)).astype(jnp.int32))

    def _bis2(_, lo_hi):
        lo, hi = lo_hi
        mid = (lo + hi) // 2
        take = _tie_cnt_le(mid) >= need
        return jnp.where(take, lo, mid + 1), jnp.where(take, mid, hi)

    _, m_star = lax.fori_loop(0, 18, _bis2,
                              (jnp.int32(0), jnp.int32(h * w - 1)))

    tmask = (yi > t_star) | (ties & (flat <= m_star))
    tmask_ref[0] = tmask.astype(jnp.int8)

    z = jnp.where(tmask, y, 0.0)
    t1 = z * _GAUSS[_R_G]
    for d in range(1, _R_G + 1):
        t1 = t1 + _GAUSS[_R_G + d] * (_shift_cols(z, d) + _shift_cols(z, -d))
    o = t1 * _GAUSS[_R_G]
    for d in range(1, _R_G + 1):
        o = o + _GAUSS[_R_G + d] * (_shift_rows(t1, d) + _shift_rows(t1, -d))
    out_ref[0] = jnp.clip(o, 0.0, 1.0)


def kernel(im1w_score):
    b, h, w, c = im1w_score.shape
    x = im1w_score.reshape(b, h, w)
    spec = pl.BlockSpec((1, h, w), lambda i: (i, 0, 0))
    out, tmask, topkv = pl.pallas_call(
        _body,
        grid=(b,),
        in_specs=[spec],
        out_specs=[spec, spec, spec],
        out_shape=[
            jax.ShapeDtypeStruct((b, h, w), jnp.float32),
            jax.ShapeDtypeStruct((b, h, w), jnp.int8),
            jax.ShapeDtypeStruct((b, h, w), jnp.float32),
        ],
        compiler_params=pltpu.CompilerParams(
            dimension_semantics=("arbitrary",)),
    )(x)
    return (out.reshape(b, h, w, c),
            tmask.reshape(b, h, w, c).astype(jnp.bool_),
            topkv.reshape(b, h, w, c))


# speedup vs baseline: 123.1154x; 1.4541x over previous
import numpy as np
import jax
import jax.numpy as jnp
from jax import lax
from jax.experimental import pallas as pl
from jax.experimental.pallas import tpu as pltpu

_K = 512
_BORDER = 8
_R_NMS = 2
_R_G = 3
_GAUSS = np.exp(-2.0 * (np.arange(-_R_G, _R_G + 1) ** 2)).astype(np.float32)
_ONE_BITS = 0x3F800000


def _body(x_ref, out_ref, tmask_ref, topkv_ref):
    h, w = x_ref.shape[1], x_ref.shape[2]
    x = x_ref[0]

    row = lax.broadcasted_iota(jnp.int32, (h, w), 0)
    col = lax.broadcasted_iota(jnp.int32, (h, w), 1)
    inb = (row >= _BORDER) & (row < h - _BORDER) & \
          (col >= _BORDER) & (col < w - _BORDER)
    xt = jnp.where(inb, x, 0.0)

    m1 = xt
    for d in range(1, _R_NMS + 1):
        m1 = jnp.maximum(m1, jnp.maximum(pltpu.roll(xt, d, 1),
                                         pltpu.roll(xt, w - d, 1)))
    mx = m1
    for d in range(1, _R_NMS + 1):
        mx = jnp.maximum(mx, jnp.maximum(pltpu.roll(m1, d, 0),
                                         pltpu.roll(m1, h - d, 0)))
    y = jnp.where(xt >= mx, xt, 0.0)
    topkv_ref[0] = y

    yi = lax.bitcast_convert_type(y, jnp.int32)

    def _cnt_ge(t):
        return jnp.sum((yi >= t).astype(jnp.int32))

    def _bis(_, carry):
        lo, hi, clo, chi = carry
        mid = (lo + hi) // 2
        c = _cnt_ge(mid)
        take = c >= _K
        return (jnp.where(take, mid, lo), jnp.where(take, hi, mid),
                jnp.where(take, c, clo), jnp.where(take, chi, c))

    t_star, _, cnt_ge_star, cnt_gt = lax.fori_loop(
        0, 30, _bis,
        (jnp.int32(0), jnp.int32(_ONE_BITS), jnp.int32(h * w), jnp.int32(0)))

    ties = yi == t_star
    flat = row * w + col

    def _tie_search():
        need = _K - cnt_gt

        def _bis2(_, lo_hi):
            lo, hi = lo_hi
            mid = (lo + hi) // 2
            take = jnp.sum((ties & (flat <= mid)).astype(jnp.int32)) >= need
            return jnp.where(take, lo, mid + 1), jnp.where(take, mid, hi)

        return lax.fori_loop(0, 18, _bis2,
                             (jnp.int32(0), jnp.int32(h * w - 1)))[1]

    m_star = lax.cond(cnt_ge_star == _K,
                      lambda: jnp.int32(h * w - 1), _tie_search)

    tmask = (yi > t_star) | (ties & (flat <= m_star))
    tmask_ref[0] = tmask.astype(jnp.int8)

    z = jnp.where(tmask, y, 0.0)
    t1 = z * _GAUSS[_R_G]
    for d in range(1, _R_G + 1):
        t1 = t1 + _GAUSS[_R_G + d] * (pltpu.roll(z, d, 1) +
                                      pltpu.roll(z, w - d, 1))
    o = t1 * _GAUSS[_R_G]
    for d in range(1, _R_G + 1):
        o = o + _GAUSS[_R_G + d] * (pltpu.roll(t1, d, 0) +
                                    pltpu.roll(t1, h - d, 0))
    out_ref[0] = jnp.clip(o, 0.0, 1.0)


def kernel(im1w_score):
    b, h, w, c = im1w_score.shape
    x = im1w_score.reshape(b, h, w)
    spec = pl.BlockSpec((1, h, w), lambda i: (i, 0, 0))
    out, tmask, topkv = pl.pallas_call(
        _body,
        grid=(b,),
        in_specs=[spec],
        out_specs=[spec, spec, spec],
        out_shape=[
            jax.ShapeDtypeStruct((b, h, w), jnp.float32),
            jax.ShapeDtypeStruct((b, h, w), jnp.int8),
            jax.ShapeDtypeStruct((b, h, w), jnp.float32),
        ],
        compiler_params=pltpu.CompilerParams(
            dimension_semantics=("arbitrary",)),
    )(x)
    return (out.reshape(b, h, w, c),
            tmask.reshape(b, h, w, c).astype(jnp.bool_),
            topkv.reshape(b, h, w, c))
